# baseline (speedup 1.0000x reference)
import functools

import jax
import jax.numpy as jnp
from jax.experimental import pallas as pl
from jax.experimental.pallas import tpu as pltpu

EPS = 1e-5
VMEM_LIMIT = 48 * 1024 * 1024
CDT = jnp.bfloat16
F32 = jnp.float32
B_P1 = 16
B_P2 = 8
B_P3 = 16


def _sums(*arrs):
    return jnp.concatenate(
        [jnp.sum(a, axis=0, keepdims=True) for a in arrs], axis=0)


def _p1_kernel(x_ref, wa_ref, wb_ref, wc_ref, ws_ref, y1_ref, r_ref, st_ref,
               xts_ref):
    b, cin, l_in = x_ref.shape
    l_out = l_in // 2
    sy = ssy = sr = ssr = 0.0
    for i in range(b):
        xts_ref[...] = x_ref[i].T
        xev = xts_ref[0::2, :].astype(CDT)
        xod = xts_ref[1::2, :].astype(CDT)
        prev = jnp.concatenate(
            [jnp.zeros((1, cin), CDT), xod[: l_out - 1, :]], axis=0)
        y1 = jnp.dot(prev, wa_ref[...], preferred_element_type=F32)
        y1 = y1 + jnp.dot(xev, wb_ref[...], preferred_element_type=F32)
        y1 = y1 + jnp.dot(xod, wc_ref[...], preferred_element_type=F32)
        r = jnp.dot(xev, ws_ref[...], preferred_element_type=F32)
        y1_ref[i] = y1.astype(y1_ref.dtype)
        r_ref[i] = r.astype(r_ref.dtype)
        sy = sy + jnp.sum(y1, axis=0, keepdims=True)
        ssy = ssy + jnp.sum(y1 * y1, axis=0, keepdims=True)
        sr = sr + jnp.sum(r, axis=0, keepdims=True)
        ssr = ssr + jnp.sum(r * r, axis=0, keepdims=True)
    st_ref[...] = jnp.concatenate([sy, ssy, sr, ssr], axis=0)


def _bn_affine(s, ssq, count, gamma, beta):
    mean = s * (1.0 / count)
    var = jnp.maximum(ssq * (1.0 / count) - mean * mean, 0.0)
    a = gamma * jax.lax.rsqrt(var + EPS)
    return a, beta - a * mean


def _p2_kernel(y1_ref, st1_ref, g_ref, bta_ref, w2_ref, y2_ref, st2_ref, *, count):
    b, l_out, c = y1_ref.shape

    s1 = jnp.sum(st1_ref[...], axis=0)
    a1, b1 = _bn_affine(s1[0:1], s1[1:2], count, g_ref[...], bta_ref[...])
    a1c, b1c = a1.astype(CDT), b1.astype(CDT)
    h3 = jnp.maximum(a1c * y1_ref[...].reshape(b * l_out, c) + b1c,
                     jnp.zeros((), CDT)).reshape(b, l_out, c)

    zrow = jnp.zeros((b, 1, c), CDT)
    h_m = h3.reshape(b * l_out, c)
    h_r = jnp.concatenate([zrow, h3[:, : l_out - 1, :]],
                          axis=1).reshape(b * l_out, c)
    h_l = jnp.concatenate([h3[:, 1:, :], zrow],
                          axis=1).reshape(b * l_out, c)
    hcat = jnp.concatenate([h_r, h_m, h_l], axis=1)
    y2 = jnp.dot(hcat, w2_ref[...].reshape(3 * c, c),
                 preferred_element_type=F32)
    y2_ref[...] = y2.astype(y2_ref.dtype).reshape(b, l_out, c)
    st2_ref[...] = _sums(y2, y2 * y2)


def _p3_kernel(y2_ref, r_ref, st1_ref, st2_ref, g_ref, bta_ref, gs_ref, bs_ref,
               out_ref, *, count):
    b = y2_ref.shape[0]
    s1 = jnp.sum(st1_ref[...], axis=0)
    s2 = jnp.sum(st2_ref[...], axis=0)
    a2, b2 = _bn_affine(s2[0:1], s2[1:2], count, g_ref[...], bta_ref[...])
    a_s, b_s = _bn_affine(s1[2:3], s1[3:4], count, gs_ref[...], bs_ref[...])
    for i in range(b):
        o = jnp.maximum(a2 * y2_ref[i].astype(F32) + b2
                        + a_s * r_ref[i].astype(F32) + b_s, 0.0)
        out_ref[i] = o.T


@jax.jit
def _run(x, w1, w2, gamma, beta, ws, gamma_s, beta_s):
    N, Cin, L = x.shape
    K = w1.shape[2]
    Cout = w1.shape[0]
    Lout = L // 2
    C2 = 2 * Cin
    count = float(N * Lout)
    B1, B2, B3 = B_P1, B_P2, B_P3
    G1, G2, G3 = N // B1, N // B2, N // B3

    w1t = jnp.transpose(w1, (2, 1, 0)).astype(CDT)
    wa, wb, wc = w1t[0], w1t[1], w1t[2]
    wsm = jnp.transpose(ws[:, :, 0], (1, 0)).astype(CDT)
    w2t = jnp.transpose(w2, (2, 1, 0)).astype(CDT)

    row = lambda v: v.astype(F32).reshape(1, Cout)
    g, bta = row(gamma), row(beta)
    gs, bs = row(gamma_s), row(beta_s)

    cparams = pltpu.CompilerParams(
        dimension_semantics=("parallel",), vmem_limit_bytes=VMEM_LIMIT)

    y1, r, st1 = pl.pallas_call(
        _p1_kernel,
        grid=(G1,),
        in_specs=[
            pl.BlockSpec((B1, Cin, L), lambda n: (n, 0, 0)),
            pl.BlockSpec((Cin, Cout), lambda n: (0, 0)),
            pl.BlockSpec((Cin, Cout), lambda n: (0, 0)),
            pl.BlockSpec((Cin, Cout), lambda n: (0, 0)),
            pl.BlockSpec((Cin, Cout), lambda n: (0, 0)),
        ],
        out_specs=[
            pl.BlockSpec((B1, Lout, Cout), lambda n: (n, 0, 0)),
            pl.BlockSpec((B1, Lout, Cout), lambda n: (n, 0, 0)),
            pl.BlockSpec((None, 4, Cout), lambda n: (n, 0, 0)),
        ],
        out_shape=[
            jax.ShapeDtypeStruct((N, Lout, Cout), CDT),
            jax.ShapeDtypeStruct((N, Lout, Cout), CDT),
            jax.ShapeDtypeStruct((G1, 4, Cout), F32),
        ],
        scratch_shapes=[pltpu.VMEM((L, Cin), F32)],
        compiler_params=cparams,
    )(x, wa, wb, wc, wsm)

    y2, st2 = pl.pallas_call(
        functools.partial(_p2_kernel, count=count),
        grid=(G2,),
        in_specs=[
            pl.BlockSpec((B2, Lout, Cout), lambda n: (n, 0, 0)),
            pl.BlockSpec((G1, 4, Cout), lambda n: (0, 0, 0)),
            pl.BlockSpec((1, Cout), lambda n: (0, 0)),
            pl.BlockSpec((1, Cout), lambda n: (0, 0)),
            pl.BlockSpec((K, Cout, Cout), lambda n: (0, 0, 0)),
        ],
        out_specs=[
            pl.BlockSpec((B2, Lout, Cout), lambda n: (n, 0, 0)),
            pl.BlockSpec((None, 2, Cout), lambda n: (n, 0, 0)),
        ],
        out_shape=[
            jax.ShapeDtypeStruct((N, Lout, Cout), CDT),
            jax.ShapeDtypeStruct((G2, 2, Cout), F32),
        ],
        compiler_params=cparams,
    )(y1, st1, g, bta, w2t)

    out = pl.pallas_call(
        functools.partial(_p3_kernel, count=count),
        grid=(G3,),
        in_specs=[
            pl.BlockSpec((B3, Lout, Cout), lambda n: (n, 0, 0)),
            pl.BlockSpec((B3, Lout, Cout), lambda n: (n, 0, 0)),
            pl.BlockSpec((G1, 4, Cout), lambda n: (0, 0, 0)),
            pl.BlockSpec((G2, 2, Cout), lambda n: (0, 0, 0)),
            pl.BlockSpec((1, Cout), lambda n: (0, 0)),
            pl.BlockSpec((1, Cout), lambda n: (0, 0)),
            pl.BlockSpec((1, Cout), lambda n: (0, 0)),
            pl.BlockSpec((1, Cout), lambda n: (0, 0)),
        ],
        out_specs=pl.BlockSpec((B3, Cout, Lout), lambda n: (n, 0, 0)),
        out_shape=jax.ShapeDtypeStruct((N, Cout, Lout), F32),
        compiler_params=cparams,
    )(y2, r, st1, st2, g, bta, gs, bs)

    return out


def kernel(x, w1, b1, w2, b2, gamma, beta, ws, bs, gamma_s, beta_s):
    return _run(x.astype(F32), w1, w2, gamma, beta, ws, gamma_s, beta_s)


# speedup vs baseline: 1.0084x; 1.0084x over previous
import functools

import jax
import jax.numpy as jnp
from jax.experimental import pallas as pl
from jax.experimental.pallas import tpu as pltpu

EPS = 1e-5
VMEM_LIMIT = 48 * 1024 * 1024
CDT = jnp.bfloat16
F32 = jnp.float32
B_P1 = 16
B_P2 = 16
B_P3 = 16


def _sums(*arrs):
    return jnp.concatenate(
        [jnp.sum(a, axis=0, keepdims=True) for a in arrs], axis=0)


def _p1_kernel(x_ref, wa_ref, wb_ref, wc_ref, ws_ref, y1_ref, r_ref, st_ref,
               xts_ref):
    b, cin, l_in = x_ref.shape
    l_out = l_in // 2
    sy = ssy = sr = ssr = 0.0
    for i in range(b):
        xts_ref[...] = x_ref[i].T
        xev = xts_ref[0::2, :].astype(CDT)
        xod = xts_ref[1::2, :].astype(CDT)
        prev = jnp.concatenate(
            [jnp.zeros((1, cin), CDT), xod[: l_out - 1, :]], axis=0)
        y1 = jnp.dot(prev, wa_ref[...], preferred_element_type=F32)
        y1 = y1 + jnp.dot(xev, wb_ref[...], preferred_element_type=F32)
        y1 = y1 + jnp.dot(xod, wc_ref[...], preferred_element_type=F32)
        r = jnp.dot(xev, ws_ref[...], preferred_element_type=F32)
        y1_ref[i] = y1.astype(y1_ref.dtype)
        r_ref[i] = r.astype(r_ref.dtype)
        sy = sy + jnp.sum(y1, axis=0, keepdims=True)
        ssy = ssy + jnp.sum(y1 * y1, axis=0, keepdims=True)
        sr = sr + jnp.sum(r, axis=0, keepdims=True)
        ssr = ssr + jnp.sum(r * r, axis=0, keepdims=True)
    st_ref[...] = jnp.concatenate([sy, ssy, sr, ssr], axis=0)


def _bn_affine(s, ssq, count, gamma, beta):
    mean = s * (1.0 / count)
    var = jnp.maximum(ssq * (1.0 / count) - mean * mean, 0.0)
    a = gamma * jax.lax.rsqrt(var + EPS)
    return a, beta - a * mean


def _p2_kernel(y1_ref, st1_ref, g_ref, bta_ref, w2_ref, y2_ref, st2_ref, *, count):
    b, l_out, c = y1_ref.shape

    s1 = jnp.sum(st1_ref[...], axis=0)
    a1, b1 = _bn_affine(s1[0:1], s1[1:2], count, g_ref[...], bta_ref[...])
    a1c, b1c = a1.astype(CDT), b1.astype(CDT)
    h3 = jnp.maximum(a1c * y1_ref[...].reshape(b * l_out, c) + b1c,
                     jnp.zeros((), CDT)).reshape(b, l_out, c)

    zrow = jnp.zeros((b, 1, c), CDT)
    h_m = h3.reshape(b * l_out, c)
    h_r = jnp.concatenate([zrow, h3[:, : l_out - 1, :]],
                          axis=1).reshape(b * l_out, c)
    h_l = jnp.concatenate([h3[:, 1:, :], zrow],
                          axis=1).reshape(b * l_out, c)
    y2 = jnp.dot(h_r, w2_ref[0], preferred_element_type=F32)
    y2 = y2 + jnp.dot(h_m, w2_ref[1], preferred_element_type=F32)
    y2 = y2 + jnp.dot(h_l, w2_ref[2], preferred_element_type=F32)
    y2_ref[...] = y2.astype(y2_ref.dtype).reshape(b, l_out, c)
    st2_ref[...] = _sums(y2, y2 * y2)


def _p3_kernel(y2_ref, r_ref, st1_ref, st2_ref, g_ref, bta_ref, gs_ref, bs_ref,
               out_ref, *, count):
    b = y2_ref.shape[0]
    s1 = jnp.sum(st1_ref[...], axis=0)
    s2 = jnp.sum(st2_ref[...], axis=0)
    a2, b2 = _bn_affine(s2[0:1], s2[1:2], count, g_ref[...], bta_ref[...])
    a_s, b_s = _bn_affine(s1[2:3], s1[3:4], count, gs_ref[...], bs_ref[...])
    for i in range(b):
        o = jnp.maximum(a2 * y2_ref[i].astype(F32) + b2
                        + a_s * r_ref[i].astype(F32) + b_s, 0.0)
        out_ref[i] = o.T


@jax.jit
def _run(x, w1, w2, gamma, beta, ws, gamma_s, beta_s):
    N, Cin, L = x.shape
    K = w1.shape[2]
    Cout = w1.shape[0]
    Lout = L // 2
    C2 = 2 * Cin
    count = float(N * Lout)
    B1, B2, B3 = B_P1, B_P2, B_P3
    G1, G2, G3 = N // B1, N // B2, N // B3

    w1t = jnp.transpose(w1, (2, 1, 0)).astype(CDT)
    wa, wb, wc = w1t[0], w1t[1], w1t[2]
    wsm = jnp.transpose(ws[:, :, 0], (1, 0)).astype(CDT)
    w2t = jnp.transpose(w2, (2, 1, 0)).astype(CDT)

    row = lambda v: v.astype(F32).reshape(1, Cout)
    g, bta = row(gamma), row(beta)
    gs, bs = row(gamma_s), row(beta_s)

    cparams = pltpu.CompilerParams(
        dimension_semantics=("parallel",), vmem_limit_bytes=VMEM_LIMIT)

    y1, r, st1 = pl.pallas_call(
        _p1_kernel,
        grid=(G1,),
        in_specs=[
            pl.BlockSpec((B1, Cin, L), lambda n: (n, 0, 0)),
            pl.BlockSpec((Cin, Cout), lambda n: (0, 0)),
            pl.BlockSpec((Cin, Cout), lambda n: (0, 0)),
            pl.BlockSpec((Cin, Cout), lambda n: (0, 0)),
            pl.BlockSpec((Cin, Cout), lambda n: (0, 0)),
        ],
        out_specs=[
            pl.BlockSpec((B1, Lout, Cout), lambda n: (n, 0, 0)),
            pl.BlockSpec((B1, Lout, Cout), lambda n: (n, 0, 0)),
            pl.BlockSpec((None, 4, Cout), lambda n: (n, 0, 0)),
        ],
        out_shape=[
            jax.ShapeDtypeStruct((N, Lout, Cout), CDT),
            jax.ShapeDtypeStruct((N, Lout, Cout), CDT),
            jax.ShapeDtypeStruct((G1, 4, Cout), F32),
        ],
        scratch_shapes=[pltpu.VMEM((L, Cin), F32)],
        compiler_params=cparams,
    )(x, wa, wb, wc, wsm)

    y2, st2 = pl.pallas_call(
        functools.partial(_p2_kernel, count=count),
        grid=(G2,),
        in_specs=[
            pl.BlockSpec((B2, Lout, Cout), lambda n: (n, 0, 0)),
            pl.BlockSpec((G1, 4, Cout), lambda n: (0, 0, 0)),
            pl.BlockSpec((1, Cout), lambda n: (0, 0)),
            pl.BlockSpec((1, Cout), lambda n: (0, 0)),
            pl.BlockSpec((K, Cout, Cout), lambda n: (0, 0, 0)),
        ],
        out_specs=[
            pl.BlockSpec((B2, Lout, Cout), lambda n: (n, 0, 0)),
            pl.BlockSpec((None, 2, Cout), lambda n: (n, 0, 0)),
        ],
        out_shape=[
            jax.ShapeDtypeStruct((N, Lout, Cout), CDT),
            jax.ShapeDtypeStruct((G2, 2, Cout), F32),
        ],
        compiler_params=cparams,
    )(y1, st1, g, bta, w2t)

    out = pl.pallas_call(
        functools.partial(_p3_kernel, count=count),
        grid=(G3,),
        in_specs=[
            pl.BlockSpec((B3, Lout, Cout), lambda n: (n, 0, 0)),
            pl.BlockSpec((B3, Lout, Cout), lambda n: (n, 0, 0)),
            pl.BlockSpec((G1, 4, Cout), lambda n: (0, 0, 0)),
            pl.BlockSpec((G2, 2, Cout), lambda n: (0, 0, 0)),
            pl.BlockSpec((1, Cout), lambda n: (0, 0)),
            pl.BlockSpec((1, Cout), lambda n: (0, 0)),
            pl.BlockSpec((1, Cout), lambda n: (0, 0)),
            pl.BlockSpec((1, Cout), lambda n: (0, 0)),
        ],
        out_specs=pl.BlockSpec((B3, Cout, Lout), lambda n: (n, 0, 0)),
        out_shape=jax.ShapeDtypeStruct((N, Cout, Lout), F32),
        compiler_params=cparams,
    )(y2, r, st1, st2, g, bta, gs, bs)

    return out


def kernel(x, w1, b1, w2, b2, gamma, beta, ws, bs, gamma_s, beta_s):
    return _run(x.astype(F32), w1, w2, gamma, beta, ws, gamma_s, beta_s)
